# Initial kernel scaffold; baseline (speedup 1.0000x reference)
import jax
import jax.numpy as jnp
from jax.experimental import pallas as pl


def kernel(x, source, position_ids, r, window_size, W_group):
    raise NotImplementedError("write your pallas kernel here")


# speedup vs baseline: 4.3394x; 4.3394x over previous
import jax
import jax.numpy as jnp
from jax.experimental import pallas as pl

_WIN = 64
_R = 8
_KEEP = _WIN - _R


def _window_kernel(x_ref, s_ref, p_ref, wt_ref, xo_ref, so_ref, po_ref):
    xw = x_ref[0]
    srcw = s_ref[0]
    posw = p_ref[0, 0]

    g = jnp.dot(xw, wt_ref[...], preferred_element_type=jnp.float32)
    n = jnp.sqrt(jnp.sum(g * g, axis=1, keepdims=True))
    gn = g / jnp.maximum(n, 1e-12)
    gnext = jnp.concatenate([gn[1:], gn[-1:]], axis=0)
    sim = jnp.sum(gn * gnext, axis=1, keepdims=True)

    iota = jax.lax.broadcasted_iota(jnp.int32, (_WIN, 1), 0)
    neginf = jnp.float32(-jnp.inf)
    cur = jnp.where(iota < _WIN - 1, sim, neginf)

    ps = jnp.zeros((_WIN, 1), dtype=jnp.bool_)
    for _ in range(_R):
        m = jnp.max(cur, axis=0, keepdims=True)
        idx = jnp.min(jnp.where(cur == m, iota, _WIN), axis=0, keepdims=True)
        ps = jnp.logical_or(ps, iota == idx)
        cur = jnp.where(jnp.abs(iota - idx) <= 1, neginf, cur)

    psi = ps.astype(jnp.int32)
    second = jnp.concatenate([jnp.zeros((1, 1), jnp.int32), psi[:-1]], axis=0)
    c = second
    for sft in (1, 2, 4, 8, 16, 32):
        c = c + jnp.concatenate(
            [jnp.zeros((sft, 1), jnp.int32), c[:-sft]], axis=0)
    keep = second == 0

    nnext = jnp.concatenate([n[1:], n[-1:]], axis=0)
    xnext = jnp.concatenate([xw[1:], xw[-1:]], axis=0)
    tot = n + nnext + 1e-8
    merged_x = jnp.where(ps, (n * xw + nnext * xnext) / tot, xw)
    snext = jnp.concatenate([srcw[1:], srcw[-1:]], axis=0)
    merged_s = srcw + jnp.where(ps, snext, 0.0)

    acc_x = jnp.zeros((_KEEP, merged_x.shape[1]), jnp.float32)
    acc_s = jnp.zeros((_KEEP, merged_s.shape[1]), jnp.float32)
    acc_p = jnp.zeros((_KEEP, 1), jnp.int32)
    for d in range(_R + 1):
        m_d = jnp.logical_and(keep, c == d)[d:d + _KEEP]
        acc_x = acc_x + jnp.where(m_d, merged_x[d:d + _KEEP], 0.0)
        acc_s = acc_s + jnp.where(m_d, merged_s[d:d + _KEEP], 0.0)
        acc_p = acc_p + jnp.where(m_d, posw[d:d + _KEEP], 0)

    xo_ref[0] = acc_x
    so_ref[0] = acc_s
    po_ref[0, 0] = acc_p


def kernel(x, source, position_ids, r, window_size, W_group):
    bsz, seq, dim = x.shape
    n_src = source.shape[2]
    nw = seq // _WIN
    wt = W_group.T
    pos4 = position_ids.reshape(bsz, nw, _WIN, 1)

    xo, so, po = pl.pallas_call(
        _window_kernel,
        grid=(bsz, nw),
        in_specs=[
            pl.BlockSpec((1, _WIN, dim), lambda b, w: (b, w, 0)),
            pl.BlockSpec((1, _WIN, n_src), lambda b, w: (b, w, 0)),
            pl.BlockSpec((1, 1, _WIN, 1), lambda b, w: (b, w, 0, 0)),
            pl.BlockSpec((dim, _WIN), lambda b, w: (0, 0)),
        ],
        out_specs=[
            pl.BlockSpec((1, _KEEP, dim), lambda b, w: (b, w, 0)),
            pl.BlockSpec((1, _KEEP, n_src), lambda b, w: (b, w, 0)),
            pl.BlockSpec((1, 1, _KEEP, 1), lambda b, w: (b, w, 0, 0)),
        ],
        out_shape=[
            jax.ShapeDtypeStruct((bsz, nw * _KEEP, dim), jnp.float32),
            jax.ShapeDtypeStruct((bsz, nw * _KEEP, n_src), jnp.float32),
            jax.ShapeDtypeStruct((bsz, nw, _KEEP, 1), jnp.int32),
        ],
    )(x, source, pos4, wt)
    return xo, so, po.reshape(bsz, nw * _KEEP)


# speedup vs baseline: 11.0576x; 2.5482x over previous
import jax
import jax.numpy as jnp
from jax.experimental import pallas as pl

_WIN = 64
_R = 8
_KEEP = _WIN - _R


def _window_kernel(x_ref, s_ref, p_ref, wt_ref, xo_ref, so_ref, po_ref):
    xw = x_ref[0]
    srcw = s_ref[0]
    posw = p_ref[0, 0]

    g = jnp.dot(xw, wt_ref[...], preferred_element_type=jnp.float32)
    n = jnp.sqrt(jnp.sum(g * g, axis=1, keepdims=True))
    gn = g / jnp.maximum(n, 1e-12)
    gnext = jnp.concatenate([gn[1:], gn[-1:]], axis=0)
    sim = jnp.sum(gn * gnext, axis=1, keepdims=True)

    kiota = jax.lax.broadcasted_iota(jnp.int32, (_WIN, 1), 0)
    neginf = jnp.float32(-jnp.inf)
    cur = jnp.where(kiota < _WIN - 1, sim, neginf)

    ps = jnp.zeros((_WIN, 1), dtype=jnp.bool_)
    for _ in range(_R):
        m = jnp.max(cur, axis=0, keepdims=True)
        idx = jnp.min(jnp.where(cur == m, kiota, _WIN), axis=0, keepdims=True)
        ps = jnp.logical_or(ps, kiota == idx)
        cur = jnp.where(jnp.abs(kiota - idx) <= 1, neginf, cur)

    psi = ps.astype(jnp.int32)
    second = jnp.concatenate([jnp.zeros((1, 1), jnp.int32), psi[:-1]], axis=0)
    c = second
    for sft in (1, 2, 4, 8, 16, 32):
        c = c + jnp.concatenate(
            [jnp.zeros((sft, 1), jnp.int32), c[:-sft]], axis=0)
    keep = second == 0
    outj = kiota - c

    jiota = jax.lax.broadcasted_iota(jnp.int32, (1, _WIN), 1)
    qt = (outj == jiota).astype(jnp.float32)

    nnext = jnp.concatenate([n[1:], n[-1:]], axis=0)
    tot = n + nnext + 1e-8
    totprev = jnp.concatenate([tot[:1], tot[:-1]], axis=0)
    wv = jnp.where(second != 0, n / totprev,
                   jnp.where(ps, n / tot, 1.0))
    wxt = qt * wv

    tdims = (((0,), (0,)), ((), ()))
    so_full = jax.lax.dot_general(qt, srcw, tdims,
                                  preferred_element_type=jnp.float32)
    xo_full = jax.lax.dot_general(wxt, xw, tdims,
                                  preferred_element_type=jnp.float32)
    xo_ref[0] = xo_full[:_KEEP]
    so_ref[0] = so_full[:_KEEP]

    acc_p = jnp.zeros((_KEEP, 1), jnp.int32)
    for d in range(_R + 1):
        m_d = jnp.logical_and(keep, c == d)[d:d + _KEEP]
        acc_p = acc_p + jnp.where(m_d, posw[d:d + _KEEP], 0)
    po_ref[0, 0] = acc_p


def kernel(x, source, position_ids, r, window_size, W_group):
    bsz, seq, dim = x.shape
    n_src = source.shape[2]
    nw = seq // _WIN
    wt = W_group.T
    pos4 = position_ids.reshape(bsz, nw, _WIN, 1)

    xo, so, po = pl.pallas_call(
        _window_kernel,
        grid=(bsz, nw),
        in_specs=[
            pl.BlockSpec((1, _WIN, dim), lambda b, w: (b, w, 0)),
            pl.BlockSpec((1, _WIN, n_src), lambda b, w: (b, w, 0)),
            pl.BlockSpec((1, 1, _WIN, 1), lambda b, w: (b, w, 0, 0)),
            pl.BlockSpec((dim, _WIN), lambda b, w: (0, 0)),
        ],
        out_specs=[
            pl.BlockSpec((1, _KEEP, dim), lambda b, w: (b, w, 0)),
            pl.BlockSpec((1, _KEEP, n_src), lambda b, w: (b, w, 0)),
            pl.BlockSpec((1, 1, _KEEP, 1), lambda b, w: (b, w, 0, 0)),
        ],
        out_shape=[
            jax.ShapeDtypeStruct((bsz, nw * _KEEP, dim), jnp.float32),
            jax.ShapeDtypeStruct((bsz, nw * _KEEP, n_src), jnp.float32),
            jax.ShapeDtypeStruct((bsz, nw, _KEEP, 1), jnp.int32),
        ],
    )(x, source, pos4, wt)
    return xo, so, po.reshape(bsz, nw * _KEEP)
